# Initial kernel scaffold; baseline (speedup 1.0000x reference)
import jax
import jax.numpy as jnp
from jax.experimental import pallas as pl


def kernel(h_oneD, h_twoD, x_dyn_oneD, x_dyn_twoD, x_static_oneD, x_static_twoD, edge_attr_flow, edge_attr_cross, edge_index_flow, edge_index_cross, params):
    raise NotImplementedError("write your pallas kernel here")


# speedup vs baseline: 1.0089x; 1.0089x over previous
import functools

import jax
import jax.numpy as jnp
from jax import lax
from jax.experimental import pallas as pl
from jax.experimental.pallas import tpu as pltpu

_N1 = 50000
_N2 = 10000
_EF = 800000
_EC = 200000
_HD = 96
_MD = 64
_HID = 128
_HEADS = 4
_C = 16
_DS = 16
_DD = 8


def _ln(x):
    mu = jnp.mean(x, axis=-1, keepdims=True)
    var = jnp.var(x, axis=-1, keepdims=True)
    return (x - mu) * lax.rsqrt(var + 1e-5)


def _gru_body(h_ref, xd_ref, msg_ref, dynW_ref, dynb_ref, Wih_ref, bih_ref,
              Whh_ref, bhh_ref, out_ref):
    h = h_ref[...]
    xd = xd_ref[...]
    msg = msg_ref[...]
    dyn = _ln(xd @ dynW_ref[...].T + dynb_ref[...])
    me = _ln(msg)
    ui = jnp.concatenate([dyn, me], axis=-1)
    gx = ui @ Wih_ref[...].T + bih_ref[...]
    gh = h @ Whh_ref[...].T + bhh_ref[...]
    xr_, xz_, xn_ = jnp.split(gx, 3, axis=-1)
    hr_, hz_, hn_ = jnp.split(gh, 3, axis=-1)
    r = jax.nn.sigmoid(xr_ + hr_)
    z = jax.nn.sigmoid(xz_ + hz_)
    n = jnp.tanh(xn_ + r * hn_)
    h_raw = (1.0 - z) * n + z * h
    out_ref[...] = _ln(h_raw)


def _gru_update(h, xd, msg, dynW, dynb, Wih, bih, Whh, bhh, block=1000):
    n = h.shape[0]
    grid = n // block
    row = lambda i: (i, 0)
    full = lambda i: (0, 0)
    vec = lambda i: (0,)
    return pl.pallas_call(
        _gru_body,
        grid=(grid,),
        in_specs=[
            pl.BlockSpec((block, _HD), row),
            pl.BlockSpec((block, _DD), row),
            pl.BlockSpec((block, _MD), row),
            pl.BlockSpec((_MD, _DD), full),
            pl.BlockSpec((_MD,), vec),
            pl.BlockSpec((3 * _HD, 2 * _MD), full),
            pl.BlockSpec((3 * _HD,), vec),
            pl.BlockSpec((3 * _HD, _HD), full),
            pl.BlockSpec((3 * _HD,), vec),
        ],
        out_specs=pl.BlockSpec((block, _HD), row),
        out_shape=jax.ShapeDtypeStruct((n, _HD), jnp.float32),
    )(h, xd, msg, dynW, dynb, Wih, bih, Whh, bhh)


def kernel(h_oneD, h_twoD, x_dyn_oneD, x_dyn_twoD, x_static_oneD,
           x_static_twoD, edge_attr_flow, edge_attr_cross, edge_index_flow,
           edge_index_cross, params):
    p = params
    h1, h2 = h_oneD, h_twoD
    s = edge_index_flow[0]
    d = edge_index_flow[1]

    W1 = p['sd_es_W1']
    W1a, W1s, W1d = W1[:, :4], W1[:, 4:4 + _DS], W1[:, 4 + _DS:]
    SA = x_static_oneD @ W1s.T + p['sd_es_b1']
    SB = x_static_oneD @ W1d.T
    wc = (p['sd_bw_W'] @ p['sd_es_W2'])[0]
    cc = p['sd_es_b2'] @ p['sd_bw_W'][0] + p['sd_bw_b'][0]
    Wg = p['sd_dg_W1']
    GA = h1 @ Wg[:, :_HD].T + p['sd_dg_b1']
    GB = h1 @ Wg[:, _HD:].T
    wg = p['sd_dg_W2'][0]
    cg = p['sd_dg_b2'][0]
    V = jax.nn.relu(h1 @ p['sd_pl_W1'].T + p['sd_pl_b1']) @ p['sd_pl_W2'].T \
        + p['sd_pl_b2']

    pre_b = SA[s] + SB[d] + edge_attr_flow @ W1a.T
    tb = jax.nn.relu(pre_b) @ wc + cc
    pre_g = GA[s] + GB[d]
    tg = jax.nn.relu(pre_g) @ wg + cg
    w = jax.nn.softplus(tb) * jax.nn.sigmoid(tg)
    msg1 = jax.ops.segment_sum(w[:, None] * V[s], d, num_segments=_N1)

    s2 = edge_index_cross[0]
    d2 = edge_index_cross[1]
    xl = h1 @ p['gat_Wl'].T
    xr = h2 @ p['gat_Wr'].T
    e = jax.nn.leaky_relu(
        (xl[s2] + xr[d2]).reshape(-1, _HEADS, _C), negative_slope=0.2)
    logits = (e * p['gat_att'][None]).sum(axis=-1)
    ex = jnp.exp(logits)
    den = jax.ops.segment_sum(ex, d2, num_segments=_N2)
    alpha = ex / (den[d2] + 1e-16)
    out = jax.ops.segment_sum(
        xl[s2].reshape(-1, _HEADS, _C) * alpha[..., None], d2,
        num_segments=_N2).reshape(_N2, _HEADS * _C)
    out = out + h2 @ p['gat_Wres'].T + p['gat_bias']
    msg2 = jax.nn.relu(out @ p['gat_ffn_W1'].T + p['gat_ffn_b1']) \
        @ p['gat_ffn_W2'].T + p['gat_ffn_b2']

    h1n = _gru_update(h1, x_dyn_oneD, msg1, p['dyn_W_oneD'], p['dyn_b_oneD'],
                      p['gru_Wih_oneD'], p['gru_bih_oneD'],
                      p['gru_Whh_oneD'], p['gru_bhh_oneD'])
    h2n = _gru_update(h2, x_dyn_twoD, msg2, p['dyn_W_twoD'], p['dyn_b_twoD'],
                      p['gru_Wih_twoD'], p['gru_bih_twoD'],
                      p['gru_Whh_twoD'], p['gru_bhh_twoD'])
    return jnp.concatenate([h1n, h2n], axis=0)


# speedup vs baseline: 1.0831x; 1.0736x over previous
import functools

import jax
import jax.numpy as jnp
from jax import lax
from jax.experimental import pallas as pl
from jax.experimental.pallas import tpu as pltpu
from jax.experimental.pallas import tpu_sc as plsc

_N1 = 50000
_N2 = 10000
_EF = 800000
_EC = 200000
_HD = 96
_MD = 64
_HID = 128
_HEADS = 4
_C = 16
_DS = 16
_DD = 8


def _ln(x):
    mu = jnp.mean(x, axis=-1, keepdims=True)
    var = jnp.var(x, axis=-1, keepdims=True)
    return (x - mu) * lax.rsqrt(var + 1e-5)


_EFP = 802816
_ROWS_PER_CORE = 25088
_HALF = 25000


def _iota16():
    return jnp.arange(16, dtype=jnp.int32)


def _full16(v):
    return jnp.full((16,), v, dtype=jnp.int32)


def _flow_aggregate(V, w_p, s_p, d_p):
    CH = 128
    PT = _EFP // 16
    NCHUNK = PT // CH
    TROWS = _ROWS_PER_CORE // 16

    mesh = plsc.VectorSubcoreMesh(core_axis_name="c", subcore_axis_name="s")

    @functools.partial(
        pl.kernel,
        out_type=jax.ShapeDtypeStruct((2 * _ROWS_PER_CORE, _MD), jnp.float32),
        mesh=mesh,
        scratch_types=[
            pltpu.VMEM((CH,), jnp.int32),
            pltpu.VMEM((CH,), jnp.int32),
            pltpu.VMEM((CH,), jnp.int32),
            pltpu.VMEM((CH,), jnp.float32),
            pltpu.VMEM((CH, _MD), jnp.float32),
            pltpu.VMEM((112, _MD), jnp.float32),
            pltpu.VMEM_SHARED((_ROWS_PER_CORE, _MD), jnp.float32),
            pltpu.SemaphoreType.DMA,
        ],
        compiler_params=pltpu.CompilerParams(use_tc_tiling_on_sc=False),
    )
    def k(V_hbm, w_hbm, s_hbm, d_hbm, out_hbm,
          sbuf, dbuf, lidx, wbuf, vrows, zbuf, acc, sem):
        c = lax.axis_index("c")
        t = lax.axis_index("s")
        base_r = c * _HALF

        def zinit(i, _):
            zbuf[i, pl.ds(0, 16)] = jnp.zeros((16,), jnp.float32)
            zbuf[i, pl.ds(16, 16)] = jnp.zeros((16,), jnp.float32)
            zbuf[i, pl.ds(32, 16)] = jnp.zeros((16,), jnp.float32)
            zbuf[i, pl.ds(48, 16)] = jnp.zeros((16,), jnp.float32)
            return 0
        lax.fori_loop(0, 112, zinit, 0)
        for kk in range(14):
            pltpu.sync_copy(zbuf, acc.at[pl.ds(t * TROWS + kk * 112, 112)])
        plsc.subcore_barrier()

        def chunk(i, _):
            base_e = t * PT + i * CH
            pltpu.sync_copy(s_hbm.at[pl.ds(base_e, CH)], sbuf)
            pltpu.sync_copy(d_hbm.at[pl.ds(base_e, CH)], dbuf)
            pltpu.sync_copy(w_hbm.at[pl.ds(base_e, CH)], wbuf)
            pltpu.async_copy(V_hbm.at[sbuf], vrows, sem).wait()
            def group(g, _):
                d16 = dbuf[pl.ds(g * 16, 16)]
                w16 = wbuf[pl.ds(g * 16, 16)]
                e16 = _iota16() + (g * 16 + base_e)
                m = (d16 >= base_r) & (d16 < base_r + _HALF)
                loc = jnp.where(m, d16 - base_r, e16 & 16383)
                wv = jnp.where(m, w16, 0.0)
                lidx[pl.ds(g * 16, 16)] = loc
                for e in range(16):
                    r = g * 16 + e
                    wvv = jnp.full((16,), wv[e], jnp.float32)
                    for j in range(4):
                        sl = pl.ds(j * 16, 16)
                        vrows[r, sl] = vrows[r, sl] * wvv
                return 0
            lax.fori_loop(0, 8, group, 0)
            pltpu.sync_copy(vrows, acc.at[lidx], add=True)
            return 0
        lax.fori_loop(0, NCHUNK, chunk, 0)
        plsc.subcore_barrier()
        pltpu.sync_copy(
            acc.at[pl.ds(t * TROWS, TROWS)],
            out_hbm.at[pl.ds(c * _ROWS_PER_CORE + t * TROWS, TROWS)])

    out = k(V, w_p, s_p, d_p)
    return jnp.concatenate(
        [out[:_HALF], out[_ROWS_PER_CORE:_ROWS_PER_CORE + _HALF]], axis=0)


def _gru_body(h_ref, xd_ref, msg_ref, dynW_ref, dynb_ref, Wih_ref, bih_ref,
              Whh_ref, bhh_ref, out_ref):
    h = h_ref[...]
    xd = xd_ref[...]
    msg = msg_ref[...]
    dyn = _ln(xd @ dynW_ref[...].T + dynb_ref[...])
    me = _ln(msg)
    ui = jnp.concatenate([dyn, me], axis=-1)
    gx = ui @ Wih_ref[...].T + bih_ref[...]
    gh = h @ Whh_ref[...].T + bhh_ref[...]
    xr_, xz_, xn_ = jnp.split(gx, 3, axis=-1)
    hr_, hz_, hn_ = jnp.split(gh, 3, axis=-1)
    r = jax.nn.sigmoid(xr_ + hr_)
    z = jax.nn.sigmoid(xz_ + hz_)
    n = jnp.tanh(xn_ + r * hn_)
    h_raw = (1.0 - z) * n + z * h
    out_ref[...] = _ln(h_raw)


def _gru_update(h, xd, msg, dynW, dynb, Wih, bih, Whh, bhh, block=1000):
    n = h.shape[0]
    grid = n // block
    row = lambda i: (i, 0)
    full = lambda i: (0, 0)
    vec = lambda i: (0,)
    return pl.pallas_call(
        _gru_body,
        grid=(grid,),
        in_specs=[
            pl.BlockSpec((block, _HD), row),
            pl.BlockSpec((block, _DD), row),
            pl.BlockSpec((block, _MD), row),
            pl.BlockSpec((_MD, _DD), full),
            pl.BlockSpec((_MD,), vec),
            pl.BlockSpec((3 * _HD, 2 * _MD), full),
            pl.BlockSpec((3 * _HD,), vec),
            pl.BlockSpec((3 * _HD, _HD), full),
            pl.BlockSpec((3 * _HD,), vec),
        ],
        out_specs=pl.BlockSpec((block, _HD), row),
        out_shape=jax.ShapeDtypeStruct((n, _HD), jnp.float32),
    )(h, xd, msg, dynW, dynb, Wih, bih, Whh, bhh)


def kernel(h_oneD, h_twoD, x_dyn_oneD, x_dyn_twoD, x_static_oneD,
           x_static_twoD, edge_attr_flow, edge_attr_cross, edge_index_flow,
           edge_index_cross, params):
    p = params
    h1, h2 = h_oneD, h_twoD
    s = edge_index_flow[0]
    d = edge_index_flow[1]

    W1 = p['sd_es_W1']
    W1a, W1s, W1d = W1[:, :4], W1[:, 4:4 + _DS], W1[:, 4 + _DS:]
    SA = x_static_oneD @ W1s.T + p['sd_es_b1']
    SB = x_static_oneD @ W1d.T
    wc = (p['sd_bw_W'] @ p['sd_es_W2'])[0]
    cc = p['sd_es_b2'] @ p['sd_bw_W'][0] + p['sd_bw_b'][0]
    Wg = p['sd_dg_W1']
    GA = h1 @ Wg[:, :_HD].T + p['sd_dg_b1']
    GB = h1 @ Wg[:, _HD:].T
    wg = p['sd_dg_W2'][0]
    cg = p['sd_dg_b2'][0]
    V = jax.nn.relu(h1 @ p['sd_pl_W1'].T + p['sd_pl_b1']) @ p['sd_pl_W2'].T \
        + p['sd_pl_b2']

    pre_b = SA[s] + SB[d] + edge_attr_flow @ W1a.T
    tb = jax.nn.relu(pre_b) @ wc + cc
    pre_g = GA[s] + GB[d]
    tg = jax.nn.relu(pre_g) @ wg + cg
    w = jax.nn.softplus(tb) * jax.nn.sigmoid(tg)
    npad = _EFP - _EF
    s_p = jnp.concatenate([s, jnp.zeros((npad,), s.dtype)])
    d_p = jnp.concatenate([d, jnp.full((npad,), 1 << 29, d.dtype)])
    w_p = jnp.concatenate([w, jnp.zeros((npad,), w.dtype)])
    msg1 = _flow_aggregate(V, w_p, s_p, d_p)

    s2 = edge_index_cross[0]
    d2 = edge_index_cross[1]
    xl = h1 @ p['gat_Wl'].T
    xr = h2 @ p['gat_Wr'].T
    e = jax.nn.leaky_relu(
        (xl[s2] + xr[d2]).reshape(-1, _HEADS, _C), negative_slope=0.2)
    logits = (e * p['gat_att'][None]).sum(axis=-1)
    ex = jnp.exp(logits)
    den = jax.ops.segment_sum(ex, d2, num_segments=_N2)
    alpha = ex / (den[d2] + 1e-16)
    out = jax.ops.segment_sum(
        xl[s2].reshape(-1, _HEADS, _C) * alpha[..., None], d2,
        num_segments=_N2).reshape(_N2, _HEADS * _C)
    out = out + h2 @ p['gat_Wres'].T + p['gat_bias']
    msg2 = jax.nn.relu(out @ p['gat_ffn_W1'].T + p['gat_ffn_b1']) \
        @ p['gat_ffn_W2'].T + p['gat_ffn_b2']

    h1n = _gru_update(h1, x_dyn_oneD, msg1, p['dyn_W_oneD'], p['dyn_b_oneD'],
                      p['gru_Wih_oneD'], p['gru_bih_oneD'],
                      p['gru_Whh_oneD'], p['gru_bhh_oneD'])
    h2n = _gru_update(h2, x_dyn_twoD, msg2, p['dyn_W_twoD'], p['dyn_b_twoD'],
                      p['gru_Wih_twoD'], p['gru_bih_twoD'],
                      p['gru_Whh_twoD'], p['gru_bhh_twoD'])
    return jnp.concatenate([h1n, h2n], axis=0)


# speedup vs baseline: 1.2425x; 1.1471x over previous
import functools

import jax
import jax.numpy as jnp
from jax import lax
from jax.experimental import pallas as pl
from jax.experimental.pallas import tpu as pltpu
from jax.experimental.pallas import tpu_sc as plsc

_N1 = 50000
_N2 = 10000
_EF = 800000
_EC = 200000
_HD = 96
_MD = 64
_HID = 128
_HEADS = 4
_C = 16
_DS = 16
_DD = 8


def _ln(x):
    mu = jnp.mean(x, axis=-1, keepdims=True)
    var = jnp.var(x, axis=-1, keepdims=True)
    return (x - mu) * lax.rsqrt(var + 1e-5)


_EFP = 802816
_ROWS_PER_CORE = 25088
_HALF = 25000


def _iota16():
    return jnp.arange(16, dtype=jnp.int32)


def _full16(v):
    return jnp.full((16,), v, dtype=jnp.int32)


_GDN = lax.GatherDimensionNumbers(
    offset_dims=(), collapsed_slice_dims=(0,), start_index_map=(0,))


def _lanesum(v):
    for m in (8, 4, 2, 1):
        idx = _iota16() ^ m
        v = v + lax.gather(v, idx[:, None], _GDN, (1,),
                           mode=lax.GatherScatterMode.PROMISE_IN_BOUNDS)
    return v


def _flow_edge_heads(P1, P2, EB_p, wc, wg, s_p, d0_p):
    CH = 112
    PT = _EFP // 32
    NCHUNK = PT // CH
    HB = _HID // 16

    mesh = plsc.VectorSubcoreMesh(core_axis_name="c", subcore_axis_name="s")

    @functools.partial(
        pl.kernel,
        out_type=[jax.ShapeDtypeStruct((_EFP,), jnp.float32),
                  jax.ShapeDtypeStruct((_EFP,), jnp.float32)],
        mesh=mesh,
        scratch_types=[
            pltpu.VMEM((CH,), jnp.int32),
            pltpu.VMEM((CH,), jnp.int32),
            pltpu.VMEM((CH, 2 * _HID), jnp.float32),
            pltpu.VMEM((CH, 2 * _HID), jnp.float32),
            pltpu.VMEM((CH, _HID), jnp.float32),
            pltpu.VMEM((_HID,), jnp.float32),
            pltpu.VMEM((_HID,), jnp.float32),
            pltpu.VMEM((CH,), jnp.float32),
            pltpu.VMEM((CH,), jnp.float32),
            pltpu.SemaphoreType.DMA,
            pltpu.SemaphoreType.DMA,
        ],
        compiler_params=pltpu.CompilerParams(use_tc_tiling_on_sc=False),
    )
    def k(P1_hbm, P2_hbm, EB_hbm, wc_hbm, wg_hbm, s_hbm, d_hbm,
          tb_hbm, tg_hbm,
          sbuf, dbuf, p1r, p2r, ebr, wcv, wgv, tbbuf, tgbuf, sem, sem2):
        wid = lax.axis_index("s") * 2 + lax.axis_index("c")
        pltpu.sync_copy(wc_hbm, wcv)
        pltpu.sync_copy(wg_hbm, wgv)

        def chunk(i, _):
            base = wid * PT + i * CH
            pltpu.sync_copy(s_hbm.at[pl.ds(base, CH)], sbuf)
            pltpu.sync_copy(d_hbm.at[pl.ds(base, CH)], dbuf)
            pltpu.sync_copy(EB_hbm.at[pl.ds(base, CH)], ebr)
            cp1 = pltpu.async_copy(P1_hbm.at[sbuf], p1r, sem)
            cp2 = pltpu.async_copy(P2_hbm.at[dbuf], p2r, sem2)
            cp1.wait()
            cp2.wait()
            wcs = [wcv[pl.ds(q * 16, 16)] for q in range(HB)]
            wgs = [wgv[pl.ds(q * 16, 16)] for q in range(HB)]

            def group(g, _):
                tbv = jnp.zeros((16,), jnp.float32)
                tgv = jnp.zeros((16,), jnp.float32)
                for e in range(16):
                    r = g * 16 + e
                    accb = jnp.zeros((16,), jnp.float32)
                    accg = jnp.zeros((16,), jnp.float32)
                    for q in range(HB):
                        sl = pl.ds(q * 16, 16)
                        sl2 = pl.ds(_HID + q * 16, 16)
                        pre_b = p1r[r, sl] + p2r[r, sl] + ebr[r, sl]
                        accb = accb + jnp.maximum(pre_b, 0.0) * wcs[q]
                        pre_g = p1r[r, sl2] + p2r[r, sl2]
                        accg = accg + jnp.maximum(pre_g, 0.0) * wgs[q]
                    lane = _iota16() == e
                    tbv = jnp.where(lane, _lanesum(accb), tbv)
                    tgv = jnp.where(lane, _lanesum(accg), tgv)
                tbbuf[pl.ds(g * 16, 16)] = tbv
                tgbuf[pl.ds(g * 16, 16)] = tgv
                return 0
            lax.fori_loop(0, CH // 16, group, 0)
            pltpu.sync_copy(tbbuf, tb_hbm.at[pl.ds(base, CH)])
            pltpu.sync_copy(tgbuf, tg_hbm.at[pl.ds(base, CH)])
            return 0
        lax.fori_loop(0, NCHUNK, chunk, 0)

    return k(P1, P2, EB_p, wc, wg, s_p, d0_p)


def _flow_aggregate(V, w_p, s_p, d_p):
    CH = 128
    PT = _EFP // 16
    NCHUNK = PT // CH
    TROWS = _ROWS_PER_CORE // 16

    mesh = plsc.VectorSubcoreMesh(core_axis_name="c", subcore_axis_name="s")

    @functools.partial(
        pl.kernel,
        out_type=jax.ShapeDtypeStruct((2 * _ROWS_PER_CORE, _MD), jnp.float32),
        mesh=mesh,
        scratch_types=[
            pltpu.VMEM((CH,), jnp.int32),
            pltpu.VMEM((CH,), jnp.int32),
            pltpu.VMEM((CH,), jnp.int32),
            pltpu.VMEM((CH,), jnp.float32),
            pltpu.VMEM((CH, _MD), jnp.float32),
            pltpu.VMEM((112, _MD), jnp.float32),
            pltpu.VMEM_SHARED((_ROWS_PER_CORE, _MD), jnp.float32),
            pltpu.SemaphoreType.DMA,
        ],
        compiler_params=pltpu.CompilerParams(use_tc_tiling_on_sc=False),
    )
    def k(V_hbm, w_hbm, s_hbm, d_hbm, out_hbm,
          sbuf, dbuf, lidx, wbuf, vrows, zbuf, acc, sem):
        c = lax.axis_index("c")
        t = lax.axis_index("s")
        base_r = c * _HALF

        def zinit(i, _):
            zbuf[i, pl.ds(0, 16)] = jnp.zeros((16,), jnp.float32)
            zbuf[i, pl.ds(16, 16)] = jnp.zeros((16,), jnp.float32)
            zbuf[i, pl.ds(32, 16)] = jnp.zeros((16,), jnp.float32)
            zbuf[i, pl.ds(48, 16)] = jnp.zeros((16,), jnp.float32)
            return 0
        lax.fori_loop(0, 112, zinit, 0)
        for kk in range(14):
            pltpu.sync_copy(zbuf, acc.at[pl.ds(t * TROWS + kk * 112, 112)])
        plsc.subcore_barrier()

        def chunk(i, _):
            base_e = t * PT + i * CH
            pltpu.sync_copy(s_hbm.at[pl.ds(base_e, CH)], sbuf)
            pltpu.sync_copy(d_hbm.at[pl.ds(base_e, CH)], dbuf)
            pltpu.sync_copy(w_hbm.at[pl.ds(base_e, CH)], wbuf)
            pltpu.async_copy(V_hbm.at[sbuf], vrows, sem).wait()
            def group(g, _):
                d16 = dbuf[pl.ds(g * 16, 16)]
                w16 = wbuf[pl.ds(g * 16, 16)]
                e16 = _iota16() + (g * 16 + base_e)
                m = (d16 >= base_r) & (d16 < base_r + _HALF)
                loc = jnp.where(m, d16 - base_r, e16 & 16383)
                wv = jnp.where(m, w16, 0.0)
                lidx[pl.ds(g * 16, 16)] = loc
                for e in range(16):
                    r = g * 16 + e
                    wvv = jnp.full((16,), wv[e], jnp.float32)
                    for j in range(4):
                        sl = pl.ds(j * 16, 16)
                        vrows[r, sl] = vrows[r, sl] * wvv
                return 0
            lax.fori_loop(0, 8, group, 0)
            pltpu.sync_copy(vrows, acc.at[lidx], add=True)
            return 0
        lax.fori_loop(0, NCHUNK, chunk, 0)
        plsc.subcore_barrier()
        pltpu.sync_copy(
            acc.at[pl.ds(t * TROWS, TROWS)],
            out_hbm.at[pl.ds(c * _ROWS_PER_CORE + t * TROWS, TROWS)])

    out = k(V, w_p, s_p, d_p)
    return jnp.concatenate(
        [out[:_HALF], out[_ROWS_PER_CORE:_ROWS_PER_CORE + _HALF]], axis=0)


def _gru_body(h_ref, xd_ref, msg_ref, dynW_ref, dynb_ref, Wih_ref, bih_ref,
              Whh_ref, bhh_ref, out_ref):
    h = h_ref[...]
    xd = xd_ref[...]
    msg = msg_ref[...]
    dyn = _ln(xd @ dynW_ref[...].T + dynb_ref[...])
    me = _ln(msg)
    ui = jnp.concatenate([dyn, me], axis=-1)
    gx = ui @ Wih_ref[...].T + bih_ref[...]
    gh = h @ Whh_ref[...].T + bhh_ref[...]
    xr_, xz_, xn_ = jnp.split(gx, 3, axis=-1)
    hr_, hz_, hn_ = jnp.split(gh, 3, axis=-1)
    r = jax.nn.sigmoid(xr_ + hr_)
    z = jax.nn.sigmoid(xz_ + hz_)
    n = jnp.tanh(xn_ + r * hn_)
    h_raw = (1.0 - z) * n + z * h
    out_ref[...] = _ln(h_raw)


def _gru_update(h, xd, msg, dynW, dynb, Wih, bih, Whh, bhh, block=1000):
    n = h.shape[0]
    grid = n // block
    row = lambda i: (i, 0)
    full = lambda i: (0, 0)
    vec = lambda i: (0,)
    return pl.pallas_call(
        _gru_body,
        grid=(grid,),
        in_specs=[
            pl.BlockSpec((block, _HD), row),
            pl.BlockSpec((block, _DD), row),
            pl.BlockSpec((block, _MD), row),
            pl.BlockSpec((_MD, _DD), full),
            pl.BlockSpec((_MD,), vec),
            pl.BlockSpec((3 * _HD, 2 * _MD), full),
            pl.BlockSpec((3 * _HD,), vec),
            pl.BlockSpec((3 * _HD, _HD), full),
            pl.BlockSpec((3 * _HD,), vec),
        ],
        out_specs=pl.BlockSpec((block, _HD), row),
        out_shape=jax.ShapeDtypeStruct((n, _HD), jnp.float32),
    )(h, xd, msg, dynW, dynb, Wih, bih, Whh, bhh)


def kernel(h_oneD, h_twoD, x_dyn_oneD, x_dyn_twoD, x_static_oneD,
           x_static_twoD, edge_attr_flow, edge_attr_cross, edge_index_flow,
           edge_index_cross, params):
    p = params
    h1, h2 = h_oneD, h_twoD
    s = edge_index_flow[0]
    d = edge_index_flow[1]

    W1 = p['sd_es_W1']
    W1a, W1s, W1d = W1[:, :4], W1[:, 4:4 + _DS], W1[:, 4 + _DS:]
    SA = x_static_oneD @ W1s.T + p['sd_es_b1']
    SB = x_static_oneD @ W1d.T
    wc = (p['sd_bw_W'] @ p['sd_es_W2'])[0]
    cc = p['sd_es_b2'] @ p['sd_bw_W'][0] + p['sd_bw_b'][0]
    Wg = p['sd_dg_W1']
    GA = h1 @ Wg[:, :_HD].T + p['sd_dg_b1']
    GB = h1 @ Wg[:, _HD:].T
    wg = p['sd_dg_W2'][0]
    cg = p['sd_dg_b2'][0]
    V = jax.nn.relu(h1 @ p['sd_pl_W1'].T + p['sd_pl_b1']) @ p['sd_pl_W2'].T \
        + p['sd_pl_b2']

    npad = _EFP - _EF
    s_p = jnp.concatenate([s, jnp.zeros((npad,), s.dtype)])
    d0_p = jnp.concatenate([d, jnp.zeros((npad,), d.dtype)])
    d_p = jnp.concatenate([d, jnp.full((npad,), 1 << 29, d.dtype)])
    P1 = jnp.concatenate([SA, GA], axis=1)
    P2 = jnp.concatenate([SB, GB], axis=1)
    EB = edge_attr_flow @ W1a.T
    EB_p = jnp.concatenate([EB, jnp.zeros((npad, _HID), EB.dtype)], axis=0)
    tb_p, tg_p = _flow_edge_heads(P1, P2, EB_p, wc, wg, s_p, d0_p)
    w_p = jax.nn.softplus(tb_p + cc) * jax.nn.sigmoid(tg_p + cg)
    msg1 = _flow_aggregate(V, w_p, s_p, d_p)

    s2 = edge_index_cross[0]
    d2 = edge_index_cross[1]
    xl = h1 @ p['gat_Wl'].T
    xr = h2 @ p['gat_Wr'].T
    e = jax.nn.leaky_relu(
        (xl[s2] + xr[d2]).reshape(-1, _HEADS, _C), negative_slope=0.2)
    logits = (e * p['gat_att'][None]).sum(axis=-1)
    ex = jnp.exp(logits)
    den = jax.ops.segment_sum(ex, d2, num_segments=_N2)
    alpha = ex / (den[d2] + 1e-16)
    out = jax.ops.segment_sum(
        xl[s2].reshape(-1, _HEADS, _C) * alpha[..., None], d2,
        num_segments=_N2).reshape(_N2, _HEADS * _C)
    out = out + h2 @ p['gat_Wres'].T + p['gat_bias']
    msg2 = jax.nn.relu(out @ p['gat_ffn_W1'].T + p['gat_ffn_b1']) \
        @ p['gat_ffn_W2'].T + p['gat_ffn_b2']

    h1n = _gru_update(h1, x_dyn_oneD, msg1, p['dyn_W_oneD'], p['dyn_b_oneD'],
                      p['gru_Wih_oneD'], p['gru_bih_oneD'],
                      p['gru_Whh_oneD'], p['gru_bhh_oneD'])
    h2n = _gru_update(h2, x_dyn_twoD, msg2, p['dyn_W_twoD'], p['dyn_b_twoD'],
                      p['gru_Wih_twoD'], p['gru_bih_twoD'],
                      p['gru_Whh_twoD'], p['gru_bhh_twoD'])
    return jnp.concatenate([h1n, h2n], axis=0)


# speedup vs baseline: 5.0030x; 4.0266x over previous
import functools

import jax
import jax.numpy as jnp
from jax import lax
from jax.experimental import pallas as pl
from jax.experimental.pallas import tpu as pltpu
from jax.experimental.pallas import tpu_sc as plsc

_N1 = 50000
_N2 = 10000
_EF = 800000
_EC = 200000
_HD = 96
_MD = 64
_HID = 128
_HEADS = 4
_C = 16
_DS = 16
_DD = 8


def _ln(x):
    mu = jnp.mean(x, axis=-1, keepdims=True)
    var = jnp.var(x, axis=-1, keepdims=True)
    return (x - mu) * lax.rsqrt(var + 1e-5)


_EFP = 802816
_ROWS_PER_CORE = 25088
_HALF = 25000


def _iota16():
    return jnp.arange(16, dtype=jnp.int32)


def _full16(v):
    return jnp.full((16,), v, dtype=jnp.int32)


_GDN = lax.GatherDimensionNumbers(
    offset_dims=(), collapsed_slice_dims=(0,), start_index_map=(0,))


def _lanesum(v):
    for m in (8, 4, 2, 1):
        idx = _iota16() ^ m
        v = v + lax.gather(v, idx[:, None], _GDN, (1,),
                           mode=lax.GatherScatterMode.PROMISE_IN_BOUNDS)
    return v


def _flow_edge_heads(P1, P2, EB_p, wc, wg, s_p, d0_p):
    CH = 112
    PT = _EFP // 32
    NCHUNK = PT // CH
    HB = _HID // 16

    mesh = plsc.VectorSubcoreMesh(core_axis_name="c", subcore_axis_name="s")

    @functools.partial(
        pl.kernel,
        out_type=[jax.ShapeDtypeStruct((_EFP,), jnp.float32),
                  jax.ShapeDtypeStruct((_EFP,), jnp.float32)],
        mesh=mesh,
        scratch_types=[
            pltpu.VMEM((CH,), jnp.int32),
            pltpu.VMEM((CH,), jnp.int32),
            pltpu.VMEM((CH, 2 * _HID), jnp.float32),
            pltpu.VMEM((CH, 2 * _HID), jnp.float32),
            pltpu.VMEM((CH, _HID), jnp.float32),
            pltpu.VMEM((_HID,), jnp.float32),
            pltpu.VMEM((_HID,), jnp.float32),
            pltpu.VMEM((CH,), jnp.float32),
            pltpu.VMEM((CH,), jnp.float32),
            pltpu.SemaphoreType.DMA,
            pltpu.SemaphoreType.DMA,
        ],
        compiler_params=pltpu.CompilerParams(use_tc_tiling_on_sc=False),
    )
    def k(P1_hbm, P2_hbm, EB_hbm, wc_hbm, wg_hbm, s_hbm, d_hbm,
          tb_hbm, tg_hbm,
          sbuf, dbuf, p1r, p2r, ebr, wcv, wgv, tbbuf, tgbuf, sem, sem2):
        wid = lax.axis_index("s") * 2 + lax.axis_index("c")
        pltpu.sync_copy(wc_hbm, wcv)
        pltpu.sync_copy(wg_hbm, wgv)

        def chunk(i, _):
            base = wid * PT + i * CH
            pltpu.sync_copy(s_hbm.at[pl.ds(base, CH)], sbuf)
            pltpu.sync_copy(d_hbm.at[pl.ds(base, CH)], dbuf)
            pltpu.sync_copy(EB_hbm.at[pl.ds(base, CH)], ebr)
            cp1 = pltpu.async_copy(P1_hbm.at[sbuf], p1r, sem)
            cp2 = pltpu.async_copy(P2_hbm.at[dbuf], p2r, sem2)
            cp1.wait()
            cp2.wait()
            wcs = [wcv[pl.ds(q * 16, 16)] for q in range(HB)]
            wgs = [wgv[pl.ds(q * 16, 16)] for q in range(HB)]

            def group(g, _):
                tbv = jnp.zeros((16,), jnp.float32)
                tgv = jnp.zeros((16,), jnp.float32)
                for e in range(16):
                    r = g * 16 + e
                    accb = jnp.zeros((16,), jnp.float32)
                    accg = jnp.zeros((16,), jnp.float32)
                    for q in range(HB):
                        sl = pl.ds(q * 16, 16)
                        sl2 = pl.ds(_HID + q * 16, 16)
                        pre_b = p1r[r, sl] + p2r[r, sl] + ebr[r, sl]
                        accb = accb + jnp.maximum(pre_b, 0.0) * wcs[q]
                        pre_g = p1r[r, sl2] + p2r[r, sl2]
                        accg = accg + jnp.maximum(pre_g, 0.0) * wgs[q]
                    lane = _iota16() == e
                    tbv = jnp.where(lane, _lanesum(accb), tbv)
                    tgv = jnp.where(lane, _lanesum(accg), tgv)
                tbbuf[pl.ds(g * 16, 16)] = tbv
                tgbuf[pl.ds(g * 16, 16)] = tgv
                return 0
            lax.fori_loop(0, CH // 16, group, 0)
            pltpu.sync_copy(tbbuf, tb_hbm.at[pl.ds(base, CH)])
            pltpu.sync_copy(tgbuf, tg_hbm.at[pl.ds(base, CH)])
            return 0
        lax.fori_loop(0, NCHUNK, chunk, 0)

    return k(P1, P2, EB_p, wc, wg, s_p, d0_p)


def _flow_aggregate(V, w_p, s_p, d_p):
    CH = 128
    PT = _EFP // 16
    NCHUNK = PT // CH
    TROWS = _ROWS_PER_CORE // 16

    mesh = plsc.VectorSubcoreMesh(core_axis_name="c", subcore_axis_name="s")

    @functools.partial(
        pl.kernel,
        out_type=jax.ShapeDtypeStruct((2 * _ROWS_PER_CORE, _MD), jnp.float32),
        mesh=mesh,
        scratch_types=[
            pltpu.VMEM((CH,), jnp.int32),
            pltpu.VMEM((CH,), jnp.int32),
            pltpu.VMEM((CH,), jnp.int32),
            pltpu.VMEM((CH,), jnp.float32),
            pltpu.VMEM((CH, _MD), jnp.float32),
            pltpu.VMEM((112, _MD), jnp.float32),
            pltpu.VMEM_SHARED((_ROWS_PER_CORE, _MD), jnp.float32),
            pltpu.SemaphoreType.DMA,
        ],
        compiler_params=pltpu.CompilerParams(use_tc_tiling_on_sc=False),
    )
    def k(V_hbm, w_hbm, s_hbm, d_hbm, out_hbm,
          sbuf, dbuf, lidx, wbuf, vrows, zbuf, acc, sem):
        c = lax.axis_index("c")
        t = lax.axis_index("s")
        base_r = c * _HALF

        def zinit(i, _):
            zbuf[i, pl.ds(0, 16)] = jnp.zeros((16,), jnp.float32)
            zbuf[i, pl.ds(16, 16)] = jnp.zeros((16,), jnp.float32)
            zbuf[i, pl.ds(32, 16)] = jnp.zeros((16,), jnp.float32)
            zbuf[i, pl.ds(48, 16)] = jnp.zeros((16,), jnp.float32)
            return 0
        lax.fori_loop(0, 112, zinit, 0)
        for kk in range(14):
            pltpu.sync_copy(zbuf, acc.at[pl.ds(t * TROWS + kk * 112, 112)])
        plsc.subcore_barrier()

        def chunk(i, _):
            base_e = t * PT + i * CH
            pltpu.sync_copy(s_hbm.at[pl.ds(base_e, CH)], sbuf)
            pltpu.sync_copy(d_hbm.at[pl.ds(base_e, CH)], dbuf)
            pltpu.sync_copy(w_hbm.at[pl.ds(base_e, CH)], wbuf)
            pltpu.async_copy(V_hbm.at[sbuf], vrows, sem).wait()
            def group(g, _):
                d16 = dbuf[pl.ds(g * 16, 16)]
                w16 = wbuf[pl.ds(g * 16, 16)]
                e16 = _iota16() + (g * 16 + base_e)
                m = (d16 >= base_r) & (d16 < base_r + _HALF)
                loc = jnp.where(m, d16 - base_r, e16 & 16383)
                wv = jnp.where(m, w16, 0.0)
                lidx[pl.ds(g * 16, 16)] = loc
                for e in range(16):
                    r = g * 16 + e
                    wvv = jnp.full((16,), wv[e], jnp.float32)
                    for j in range(4):
                        sl = pl.ds(j * 16, 16)
                        vrows[r, sl] = vrows[r, sl] * wvv
                return 0
            lax.fori_loop(0, 8, group, 0)
            pltpu.sync_copy(vrows, acc.at[lidx], add=True)
            return 0
        lax.fori_loop(0, NCHUNK, chunk, 0)
        plsc.subcore_barrier()
        pltpu.sync_copy(
            acc.at[pl.ds(t * TROWS, TROWS)],
            out_hbm.at[pl.ds(c * _ROWS_PER_CORE + t * TROWS, TROWS)])

    out = k(V, w_p, s_p, d_p)
    return jnp.concatenate(
        [out[:_HALF], out[_ROWS_PER_CORE:_ROWS_PER_CORE + _HALF]], axis=0)


_ECP = 204800
_NR2 = 10240


def _gat_edge_pass1(xl, xr, att_f, s2_p, d2_p):
    CH = 128
    PT = _ECP // 32
    NCHUNK = PT // CH
    mesh = plsc.VectorSubcoreMesh(core_axis_name="c", subcore_axis_name="s")

    @functools.partial(
        pl.kernel,
        out_type=[jax.ShapeDtypeStruct((2 * _NR2, 16), jnp.float32),
                  jax.ShapeDtypeStruct((4 * _ECP,), jnp.float32)],
        mesh=mesh,
        scratch_types=[
            pltpu.VMEM((CH,), jnp.int32),
            pltpu.VMEM((CH,), jnp.int32),
            pltpu.VMEM((CH, 64), jnp.float32),
            pltpu.VMEM((CH, 64), jnp.float32),
            pltpu.VMEM((CH, 16), jnp.float32),
            pltpu.VMEM((4 * CH,), jnp.float32),
            pltpu.VMEM((64,), jnp.float32),
            pltpu.VMEM((128, 16), jnp.float32),
            pltpu.VMEM_SHARED((_NR2, 16), jnp.float32),
            pltpu.SemaphoreType.DMA,
            pltpu.SemaphoreType.DMA,
        ],
        compiler_params=pltpu.CompilerParams(use_tc_tiling_on_sc=False),
    )
    def k(xl_hbm, xr_hbm, att_hbm, s_hbm, d_hbm, den_hbm, ex_hbm,
          sbuf, dbuf, xlr, xrr, exrow, ex4, attv, zbuf, acc, sem, sem2):
        c = lax.axis_index("c")
        t = lax.axis_index("s")
        wid = t * 2 + c
        pltpu.sync_copy(att_hbm, attv)

        def zinit(i, _):
            zbuf[i, pl.ds(0, 16)] = jnp.zeros((16,), jnp.float32)
            return 0
        lax.fori_loop(0, 128, zinit, 0)
        for kk in range(5):
            pltpu.sync_copy(zbuf, acc.at[pl.ds(t * 640 + kk * 128, 128)])
        plsc.subcore_barrier()

        atts = [attv[pl.ds(h * 16, 16)] for h in range(4)]

        def chunk(i, _):
            base = wid * PT + i * CH
            pltpu.sync_copy(s_hbm.at[pl.ds(base, CH)], sbuf)
            pltpu.sync_copy(d_hbm.at[pl.ds(base, CH)], dbuf)
            cp1 = pltpu.async_copy(xl_hbm.at[sbuf], xlr, sem)
            cp2 = pltpu.async_copy(xr_hbm.at[dbuf], xrr, sem2)
            cp1.wait()
            cp2.wait()

            def group(g, _):
                exvs = [jnp.zeros((16,), jnp.float32) for _ in range(4)]
                e16 = _iota16() + (g * 16 + base)
                mvalf = jnp.where(e16 < _EC, 1.0, 0.0)
                for e in range(16):
                    r = g * 16 + e
                    lane = _iota16() == e
                    mm = jnp.full((16,), mvalf[e], jnp.float32)
                    rowv = jnp.zeros((16,), jnp.float32)
                    for h in range(4):
                        sl = pl.ds(h * 16, 16)
                        z = xlr[r, sl] + xrr[r, sl]
                        y = jnp.maximum(z, 0.2 * z) * atts[h]
                        ex_h = jnp.exp(_lanesum(y)) * mm
                        rowv = jnp.where(_iota16() == h, ex_h, rowv)
                        exvs[h] = jnp.where(lane, ex_h, exvs[h])
                    exrow[r, pl.ds(0, 16)] = rowv
                for h in range(4):
                    ex4[pl.ds(h * CH + g * 16, 16)] = exvs[h]
                return 0
            lax.fori_loop(0, CH // 16, group, 0)
            pltpu.sync_copy(exrow, acc.at[dbuf], add=True)
            for h in range(4):
                pltpu.sync_copy(ex4.at[pl.ds(h * CH, CH)],
                                ex_hbm.at[pl.ds(h * _ECP + base, CH)])
            return 0
        lax.fori_loop(0, NCHUNK, chunk, 0)
        plsc.subcore_barrier()
        pltpu.sync_copy(acc.at[pl.ds(t * 640, 640)],
                        den_hbm.at[pl.ds(c * _NR2 + t * 640, 640)])

    return k(xl, xr, att_f, s2_p, d2_p)


def _gat_edge_pass2(xl, inv_den, exT, s2_p, d2_p):
    CH = 128
    PT = _ECP // 32
    NCHUNK = PT // CH
    mesh = plsc.VectorSubcoreMesh(core_axis_name="c", subcore_axis_name="s")

    @functools.partial(
        pl.kernel,
        out_type=jax.ShapeDtypeStruct((2 * _NR2, 64), jnp.float32),
        mesh=mesh,
        scratch_types=[
            pltpu.VMEM((CH,), jnp.int32),
            pltpu.VMEM((CH,), jnp.int32),
            pltpu.VMEM((CH, 64), jnp.float32),
            pltpu.VMEM((CH, 16), jnp.float32),
            pltpu.VMEM((4 * CH,), jnp.float32),
            pltpu.VMEM((128, 64), jnp.float32),
            pltpu.VMEM_SHARED((_NR2, 64), jnp.float32),
            pltpu.SemaphoreType.DMA,
            pltpu.SemaphoreType.DMA,
        ],
        compiler_params=pltpu.CompilerParams(use_tc_tiling_on_sc=False),
    )
    def k(xl_hbm, inv_hbm, ex_hbm, s_hbm, d_hbm, out_hbm,
          sbuf, dbuf, xlr, invr, ex4, zbuf, acc, sem, sem2):
        c = lax.axis_index("c")
        t = lax.axis_index("s")
        wid = t * 2 + c

        def zinit(i, _):
            for q in range(4):
                zbuf[i, pl.ds(q * 16, 16)] = jnp.zeros((16,), jnp.float32)
            return 0
        lax.fori_loop(0, 128, zinit, 0)
        for kk in range(5):
            pltpu.sync_copy(zbuf, acc.at[pl.ds(t * 640 + kk * 128, 128)])
        plsc.subcore_barrier()

        def chunk(i, _):
            base = wid * PT + i * CH
            pltpu.sync_copy(s_hbm.at[pl.ds(base, CH)], sbuf)
            pltpu.sync_copy(d_hbm.at[pl.ds(base, CH)], dbuf)
            for h in range(4):
                pltpu.sync_copy(ex_hbm.at[pl.ds(h * _ECP + base, CH)],
                                ex4.at[pl.ds(h * CH, CH)])
            cp1 = pltpu.async_copy(xl_hbm.at[sbuf], xlr, sem)
            cp2 = pltpu.async_copy(inv_hbm.at[dbuf], invr, sem2)
            cp1.wait()
            cp2.wait()

            def group(g, _):
                exh = [ex4[pl.ds(h * CH + g * 16, 16)] for h in range(4)]
                for e in range(16):
                    r = g * 16 + e
                    erow = jnp.zeros((16,), jnp.float32)
                    for h in range(4):
                        sp = lax.gather(
                            exh[h], _full16(e)[:, None], _GDN, (1,),
                            mode=lax.GatherScatterMode.PROMISE_IN_BOUNDS)
                        erow = jnp.where(_iota16() == h, sp, erow)
                    arow = erow * invr[r, pl.ds(0, 16)]
                    for q in range(4):
                        aq = lax.gather(
                            arow, _full16(q)[:, None], _GDN, (1,),
                            mode=lax.GatherScatterMode.PROMISE_IN_BOUNDS)
                        sl = pl.ds(q * 16, 16)
                        xlr[r, sl] = xlr[r, sl] * aq
                return 0
            lax.fori_loop(0, CH // 16, group, 0)
            pltpu.sync_copy(xlr, acc.at[dbuf], add=True)
            return 0
        lax.fori_loop(0, NCHUNK, chunk, 0)
        plsc.subcore_barrier()
        pltpu.sync_copy(acc.at[pl.ds(t * 640, 640)],
                        out_hbm.at[pl.ds(c * _NR2 + t * 640, 640)])

    return k(xl, inv_den, exT, s2_p, d2_p)


def _gru_body(h_ref, xd_ref, msg_ref, dynW_ref, dynb_ref, Wih_ref, bih_ref,
              Whh_ref, bhh_ref, out_ref):
    h = h_ref[...]
    xd = xd_ref[...]
    msg = msg_ref[...]
    dyn = _ln(xd @ dynW_ref[...].T + dynb_ref[...])
    me = _ln(msg)
    ui = jnp.concatenate([dyn, me], axis=-1)
    gx = ui @ Wih_ref[...].T + bih_ref[...]
    gh = h @ Whh_ref[...].T + bhh_ref[...]
    xr_, xz_, xn_ = jnp.split(gx, 3, axis=-1)
    hr_, hz_, hn_ = jnp.split(gh, 3, axis=-1)
    r = jax.nn.sigmoid(xr_ + hr_)
    z = jax.nn.sigmoid(xz_ + hz_)
    n = jnp.tanh(xn_ + r * hn_)
    h_raw = (1.0 - z) * n + z * h
    out_ref[...] = _ln(h_raw)


def _gru_update(h, xd, msg, dynW, dynb, Wih, bih, Whh, bhh, block=1000):
    n = h.shape[0]
    grid = n // block
    row = lambda i: (i, 0)
    full = lambda i: (0, 0)
    vec = lambda i: (0,)
    return pl.pallas_call(
        _gru_body,
        grid=(grid,),
        in_specs=[
            pl.BlockSpec((block, _HD), row),
            pl.BlockSpec((block, _DD), row),
            pl.BlockSpec((block, _MD), row),
            pl.BlockSpec((_MD, _DD), full),
            pl.BlockSpec((_MD,), vec),
            pl.BlockSpec((3 * _HD, 2 * _MD), full),
            pl.BlockSpec((3 * _HD,), vec),
            pl.BlockSpec((3 * _HD, _HD), full),
            pl.BlockSpec((3 * _HD,), vec),
        ],
        out_specs=pl.BlockSpec((block, _HD), row),
        out_shape=jax.ShapeDtypeStruct((n, _HD), jnp.float32),
    )(h, xd, msg, dynW, dynb, Wih, bih, Whh, bhh)


def kernel(h_oneD, h_twoD, x_dyn_oneD, x_dyn_twoD, x_static_oneD,
           x_static_twoD, edge_attr_flow, edge_attr_cross, edge_index_flow,
           edge_index_cross, params):
    p = params
    h1, h2 = h_oneD, h_twoD
    s = edge_index_flow[0]
    d = edge_index_flow[1]

    W1 = p['sd_es_W1']
    W1a, W1s, W1d = W1[:, :4], W1[:, 4:4 + _DS], W1[:, 4 + _DS:]
    SA = x_static_oneD @ W1s.T + p['sd_es_b1']
    SB = x_static_oneD @ W1d.T
    wc = (p['sd_bw_W'] @ p['sd_es_W2'])[0]
    cc = p['sd_es_b2'] @ p['sd_bw_W'][0] + p['sd_bw_b'][0]
    Wg = p['sd_dg_W1']
    GA = h1 @ Wg[:, :_HD].T + p['sd_dg_b1']
    GB = h1 @ Wg[:, _HD:].T
    wg = p['sd_dg_W2'][0]
    cg = p['sd_dg_b2'][0]
    V = jax.nn.relu(h1 @ p['sd_pl_W1'].T + p['sd_pl_b1']) @ p['sd_pl_W2'].T \
        + p['sd_pl_b2']

    npad = _EFP - _EF
    s_p = jnp.concatenate([s, jnp.zeros((npad,), s.dtype)])
    d0_p = jnp.concatenate([d, jnp.zeros((npad,), d.dtype)])
    d_p = jnp.concatenate([d, jnp.full((npad,), 1 << 29, d.dtype)])
    P1 = jnp.concatenate([SA, GA], axis=1)
    P2 = jnp.concatenate([SB, GB], axis=1)
    EB = edge_attr_flow @ W1a.T
    EB_p = jnp.concatenate([EB, jnp.zeros((npad, _HID), EB.dtype)], axis=0)
    tb_p, tg_p = _flow_edge_heads(P1, P2, EB_p, wc, wg, s_p, d0_p)
    w_p = jax.nn.softplus(tb_p + cc) * jax.nn.sigmoid(tg_p + cg)
    msg1 = _flow_aggregate(V, w_p, s_p, d_p)

    s2 = edge_index_cross[0]
    d2 = edge_index_cross[1]
    xl = h1 @ p['gat_Wl'].T
    xr = h2 @ p['gat_Wr'].T
    np2 = _ECP - _EC
    s2_p = jnp.concatenate([s2, jnp.zeros((np2,), s2.dtype)])
    d2_p = jnp.concatenate([d2, jnp.zeros((np2,), d2.dtype)])
    att_f = p['gat_att'].reshape(_HEADS * _C)
    den2, exT = _gat_edge_pass1(xl, xr, att_f, s2_p, d2_p)
    inv_den = 1.0 / (den2[:_NR2] + den2[_NR2:] + 1e-16)
    outp = _gat_edge_pass2(xl, inv_den, exT, s2_p, d2_p)
    out = outp[:_N2] + outp[_NR2:_NR2 + _N2]
    out = out + h2 @ p['gat_Wres'].T + p['gat_bias']
    msg2 = jax.nn.relu(out @ p['gat_ffn_W1'].T + p['gat_ffn_b1']) \
        @ p['gat_ffn_W2'].T + p['gat_ffn_b2']

    h1n = _gru_update(h1, x_dyn_oneD, msg1, p['dyn_W_oneD'], p['dyn_b_oneD'],
                      p['gru_Wih_oneD'], p['gru_bih_oneD'],
                      p['gru_Whh_oneD'], p['gru_bhh_oneD'])
    h2n = _gru_update(h2, x_dyn_twoD, msg2, p['dyn_W_twoD'], p['dyn_b_twoD'],
                      p['gru_Wih_twoD'], p['gru_bih_twoD'],
                      p['gru_Whh_twoD'], p['gru_bhh_twoD'])
    return jnp.concatenate([h1n, h2n], axis=0)
